# Initial kernel scaffold; baseline (speedup 1.0000x reference)
import jax
import jax.numpy as jnp
from jax.experimental import pallas as pl


def kernel(positions, W1, b1, W2, b2, Wp, bp, supernode_indices):
    raise NotImplementedError("write your pallas kernel here")


# speedup vs baseline: 6.1754x; 6.1754x over previous
import numpy as np

import jax
import jax.numpy as jnp
from jax.experimental import pallas as pl
from jax.experimental.pallas import tpu as pltpu

N_POINTS = 50000
N_SUPER = 1024
K = 32
H = 512
NDIM = 3
L = 128
C = 392
NP = C * L
BS = 64
PADVAL = 1.0e4
BIG = 1.0e30
BIGI = 1.0e9


def _build_consts():
    cst = np.zeros((16, H), np.float32)
    lane = np.arange(H)
    half = 64
    om = 1.0 / (10000.0 ** (np.arange(half, dtype=np.float32) / half))
    for d in range(4):
        cst[d, d * 128:(d + 1) * 128] = 1.0
    within = lane % 128
    cst[4] = om.astype(np.float32)[within % half]
    cst[5] = np.where(within < half, 0.0, np.pi / 2).astype(np.float32)
    half2 = 85
    om2 = 1.0 / (10000.0 ** (np.arange(half2, dtype=np.float32) / half2))
    for d in range(3):
        cst[6 + d, d * 170:(d + 1) * 170] = 1.0
    e = lane % 170
    valid = lane < 510
    cst[9] = np.where(valid, om2.astype(np.float32)[(e % 170) % half2], 0.0)
    cst[10] = np.where(valid & (e >= half2), np.pi / 2, 0.0).astype(np.float32)
    cst[11] = valid.astype(np.float32)
    return cst


_CONSTS = _build_consts()


def _knn_kernel(q_ref, p3_ref, b_ref, idx_ref, oh_ref):
    q = q_ref[...]
    d2 = None
    for d in range(NDIM):
        t = p3_ref[d][None, :, :] - q[:, d].reshape(BS, 1, 1)
        t = t * t
        d2 = t if d2 is None else d2 + t
    m_chunk = jnp.min(d2, axis=2)
    iota_c = jax.lax.broadcasted_iota(jnp.int32, (BS, C), 1).astype(jnp.float32)
    csel = []
    for k in range(K):
        m = jnp.min(m_chunk, axis=1, keepdims=True)
        cstar = jnp.min(jnp.where(m_chunk == m, iota_c, BIGI), axis=1,
                        keepdims=True)
        oh = iota_c == cstar
        oh_ref[:, k, :] = oh.astype(jnp.float32)
        csel.append(cstar)
        m_chunk = jnp.where(oh, BIG, m_chunk)
    csel = jnp.concatenate(csel, axis=1)
    cand = jnp.dot(oh_ref[...].reshape(BS * K, C), b_ref[...],
                   preferred_element_type=jnp.float32)
    d2c = None
    for d in range(NDIM):
        cd = cand[:, d * L:(d + 1) * L].reshape(BS, K, L)
        t = cd - q[:, d].reshape(BS, 1, 1)
        t = t * t
        d2c = t if d2c is None else d2c + t
    d2c = d2c.reshape(BS, K * L)
    gidx = (csel[:, :, None] * L
            + jax.lax.broadcasted_iota(jnp.int32, (BS, K, L), 2).astype(jnp.float32)
            ).reshape(BS, K * L)
    cols = []
    for k in range(K):
        m = jnp.min(d2c, axis=1, keepdims=True)
        hit = d2c == m
        g = jnp.min(jnp.where(hit, gidx, BIGI), axis=1, keepdims=True)
        cols.append(g)
        d2c = jnp.where(hit & (gidx == g), BIG, d2c)
    idx_ref[...] = jnp.concatenate(cols, axis=1).astype(jnp.int32)


def _mlp_kernel(np_ref, q_ref, w1_ref, b1_ref, w2_ref, b2_ref, wp_ref, bp_ref,
                cst_ref, out_ref):
    cst = cst_ref[...]
    q = q_ref[...]
    neigh = np_ref[...]
    rel = neigh - q[:, None, :]
    dist = jnp.sqrt(jnp.sum(rel * rel, axis=-1, keepdims=True) + 1e-12)
    rf = jnp.concatenate([rel, dist], axis=-1).reshape(BS * K, 4)
    c_rep = None
    for d in range(4):
        t = rf[:, d:d + 1] * cst[d:d + 1, :]
        c_rep = t if c_rep is None else c_rep + t
    emb = jnp.sin(c_rep * cst[4:5, :] + cst[5:6, :])
    x = jnp.dot(emb, w1_ref[...], preferred_element_type=jnp.float32) + b1_ref[...]
    hmid = x * 0.5 * (1.0 + jax.lax.erf(x * np.float32(1.0 / np.sqrt(2.0))))
    msg = jnp.dot(hmid, w2_ref[...], preferred_element_type=jnp.float32) + b2_ref[...]
    agg = jnp.mean(msg.reshape(BS, K, H), axis=1)
    c2 = None
    for d in range(3):
        t = q[:, d:d + 1] * cst[6 + d:7 + d, :]
        c2 = t if c2 is None else c2 + t
    emb2 = jnp.sin(c2 * cst[9:10, :] + cst[10:11, :]) * cst[11:12, :]
    out_ref[...] = (jnp.dot(agg + emb2, wp_ref[...],
                            preferred_element_type=jnp.float32) + bp_ref[...])


def kernel(positions, W1, b1, W2, b2, Wp, bp, supernode_indices):
    pos_pad = jnp.concatenate(
        [positions, jnp.full((NP - N_POINTS, NDIM), PADVAL, jnp.float32)], axis=0)
    p3 = pos_pad.T.reshape(NDIM, C, L)
    bmat = pos_pad.reshape(C, L, NDIM).transpose(0, 2, 1).reshape(C, NDIM * L)
    q = jnp.take(positions, supernode_indices, axis=0)

    idx = pl.pallas_call(
        _knn_kernel,
        grid=(N_SUPER // BS,),
        in_specs=[
            pl.BlockSpec((BS, NDIM), lambda i: (i, 0)),
            pl.BlockSpec((NDIM, C, L), lambda i: (0, 0, 0)),
            pl.BlockSpec((C, NDIM * L), lambda i: (0, 0)),
        ],
        out_specs=pl.BlockSpec((BS, K), lambda i: (i, 0)),
        out_shape=jax.ShapeDtypeStruct((N_SUPER, K), jnp.int32),
        scratch_shapes=[pltpu.VMEM((BS, K, C), jnp.float32)],
        compiler_params=pltpu.CompilerParams(dimension_semantics=("parallel",)),
    )(q, p3, bmat)

    neigh = jnp.take(positions, idx.reshape(-1), axis=0).reshape(N_SUPER, K, NDIM)

    cst = jnp.asarray(_CONSTS)
    out = pl.pallas_call(
        _mlp_kernel,
        grid=(N_SUPER // BS,),
        in_specs=[
            pl.BlockSpec((BS, K, NDIM), lambda i: (i, 0, 0)),
            pl.BlockSpec((BS, NDIM), lambda i: (i, 0)),
            pl.BlockSpec((H, H), lambda i: (0, 0)),
            pl.BlockSpec((1, H), lambda i: (0, 0)),
            pl.BlockSpec((H, H), lambda i: (0, 0)),
            pl.BlockSpec((1, H), lambda i: (0, 0)),
            pl.BlockSpec((H, H), lambda i: (0, 0)),
            pl.BlockSpec((1, H), lambda i: (0, 0)),
            pl.BlockSpec((16, H), lambda i: (0, 0)),
        ],
        out_specs=pl.BlockSpec((BS, H), lambda i: (i, 0)),
        out_shape=jax.ShapeDtypeStruct((N_SUPER, H), jnp.float32),
        compiler_params=pltpu.CompilerParams(dimension_semantics=("parallel",)),
    )(neigh, q, W1, b1.reshape(1, H), W2, b2.reshape(1, H), Wp, bp.reshape(1, H), cst)

    return out[None, :, :]


# speedup vs baseline: 6.2612x; 1.0139x over previous
import numpy as np

import jax
import jax.numpy as jnp
from jax.experimental import pallas as pl
from jax.experimental.pallas import tpu as pltpu
from jax.experimental.pallas import tpu_sc as plsc

N_POINTS = 50000
N_SUPER = 1024
K = 32
H = 512
NDIM = 3
L = 128
C = 392
NP = C * L
BS = 64
PADVAL = 1.0e4
BIG = 1.0e30
BIGI = 1.0e9


def _build_consts():
    cst = np.zeros((16, H), np.float32)
    lane = np.arange(H)
    half = 64
    om = 1.0 / (10000.0 ** (np.arange(half, dtype=np.float32) / half))
    for d in range(4):
        cst[d, d * 128:(d + 1) * 128] = 1.0
    within = lane % 128
    cst[4] = om.astype(np.float32)[within % half]
    cst[5] = np.where(within < half, 0.0, np.pi / 2).astype(np.float32)
    half2 = 85
    om2 = 1.0 / (10000.0 ** (np.arange(half2, dtype=np.float32) / half2))
    for d in range(3):
        cst[6 + d, d * 170:(d + 1) * 170] = 1.0
    e = lane % 170
    valid = lane < 510
    cst[9] = np.where(valid, om2.astype(np.float32)[(e % 170) % half2], 0.0)
    cst[10] = np.where(valid & (e >= half2), np.pi / 2, 0.0).astype(np.float32)
    cst[11] = valid.astype(np.float32)
    return cst


_CONSTS = _build_consts()


def _sc_gather_rows(data, indices, window):
    num_idx = indices.shape[0]
    dim = data.shape[1]
    mesh = plsc.VectorSubcoreMesh(core_axis_name="c", subcore_axis_name="s")
    idx2 = indices.reshape(num_idx // window, window)

    @pl.kernel(out_type=jax.ShapeDtypeStruct((num_idx, dim), data.dtype),
               mesh=mesh)
    def _gather(x_hbm, i_hbm, o_hbm):
        def body(i_vmem, o_vmem):
            pltpu.sync_copy(x_hbm.at[i_vmem.at[0]], o_vmem)

        pltpu.emit_pipeline(
            body,
            grid=(num_idx // window,),
            in_specs=[pl.BlockSpec((1, window), index_map=lambda i: (i, 0))],
            out_specs=[pl.BlockSpec((window, dim), index_map=lambda i: (i, 0))],
            core_axis_name=("c", "s"),
            dimension_semantics=(pltpu.PARALLEL,),
        )(i_hbm, o_hbm)

    return _gather(data, idx2)


def _knn_kernel(q_ref, p3_ref, b_ref, idx_ref, oh_ref):
    q = q_ref[...]
    d2 = None
    for d in range(NDIM):
        t = p3_ref[d][None, :, :] - q[:, d].reshape(BS, 1, 1)
        t = t * t
        d2 = t if d2 is None else d2 + t
    m_chunk = jnp.min(d2, axis=2)
    iota_c = jax.lax.broadcasted_iota(jnp.int32, (BS, C), 1).astype(jnp.float32)
    csel = []
    for k in range(K):
        m = jnp.min(m_chunk, axis=1, keepdims=True)
        cstar = jnp.min(jnp.where(m_chunk == m, iota_c, BIGI), axis=1,
                        keepdims=True)
        oh = iota_c == cstar
        oh_ref[:, k, :] = oh.astype(jnp.float32)
        csel.append(cstar)
        m_chunk = jnp.where(oh, BIG, m_chunk)
    csel = jnp.concatenate(csel, axis=1)
    cand = jnp.dot(oh_ref[...].reshape(BS * K, C), b_ref[...],
                   preferred_element_type=jnp.float32)
    d2c = None
    for d in range(NDIM):
        cd = cand[:, d * L:(d + 1) * L].reshape(BS, K, L)
        t = cd - q[:, d].reshape(BS, 1, 1)
        t = t * t
        d2c = t if d2c is None else d2c + t
    d2c = d2c.reshape(BS, K * L)
    gidx = (csel[:, :, None] * L
            + jax.lax.broadcasted_iota(jnp.int32, (BS, K, L), 2).astype(jnp.float32)
            ).reshape(BS, K * L)
    cols = []
    for k in range(K):
        m = jnp.min(d2c, axis=1, keepdims=True)
        hit = d2c == m
        g = jnp.min(jnp.where(hit, gidx, BIGI), axis=1, keepdims=True)
        cols.append(g)
        d2c = jnp.where(hit & (gidx == g), BIG, d2c)
    idx_ref[...] = jnp.concatenate(cols, axis=1).astype(jnp.int32)


def _mlp_kernel(np_ref, q_ref, w1_ref, b1_ref, w2_ref, b2_ref, wp_ref, bp_ref,
                cst_ref, out_ref):
    cst = cst_ref[...]
    q = q_ref[...]
    neigh = np_ref[...]
    rel = neigh - q[:, None, :]
    dist = jnp.sqrt(jnp.sum(rel * rel, axis=-1, keepdims=True) + 1e-12)
    rf = jnp.concatenate([rel, dist], axis=-1).reshape(BS * K, 4)
    c_rep = None
    for d in range(4):
        t = rf[:, d:d + 1] * cst[d:d + 1, :]
        c_rep = t if c_rep is None else c_rep + t
    emb = jnp.sin(c_rep * cst[4:5, :] + cst[5:6, :])
    x = jnp.dot(emb, w1_ref[...], preferred_element_type=jnp.float32) + b1_ref[...]
    hmid = x * 0.5 * (1.0 + jax.lax.erf(x * np.float32(1.0 / np.sqrt(2.0))))
    msg = jnp.dot(hmid, w2_ref[...], preferred_element_type=jnp.float32) + b2_ref[...]
    agg = jnp.mean(msg.reshape(BS, K, H), axis=1)
    c2 = None
    for d in range(3):
        t = q[:, d:d + 1] * cst[6 + d:7 + d, :]
        c2 = t if c2 is None else c2 + t
    emb2 = jnp.sin(c2 * cst[9:10, :] + cst[10:11, :]) * cst[11:12, :]
    out_ref[...] = (jnp.dot(agg + emb2, wp_ref[...],
                            preferred_element_type=jnp.float32) + bp_ref[...])


def kernel(positions, W1, b1, W2, b2, Wp, bp, supernode_indices):
    pos_pad = jnp.concatenate(
        [positions, jnp.full((NP - N_POINTS, NDIM), PADVAL, jnp.float32)], axis=0)
    p3 = pos_pad.T.reshape(NDIM, C, L)
    bmat = pos_pad.reshape(C, L, NDIM).transpose(0, 2, 1).reshape(C, NDIM * L)
    pos128 = jnp.pad(positions, ((0, 0), (0, 128 - NDIM)))
    q = _sc_gather_rows(pos128, supernode_indices, 32)[:, :NDIM]

    idx = pl.pallas_call(
        _knn_kernel,
        grid=(N_SUPER // BS,),
        in_specs=[
            pl.BlockSpec((BS, NDIM), lambda i: (i, 0)),
            pl.BlockSpec((NDIM, C, L), lambda i: (0, 0, 0)),
            pl.BlockSpec((C, NDIM * L), lambda i: (0, 0)),
        ],
        out_specs=pl.BlockSpec((BS, K), lambda i: (i, 0)),
        out_shape=jax.ShapeDtypeStruct((N_SUPER, K), jnp.int32),
        scratch_shapes=[pltpu.VMEM((BS, K, C), jnp.float32)],
        compiler_params=pltpu.CompilerParams(dimension_semantics=("parallel",)),
    )(q, p3, bmat)

    neigh = _sc_gather_rows(pos128, idx.reshape(-1), 128)[:, :NDIM].reshape(
        N_SUPER, K, NDIM)

    cst = jnp.asarray(_CONSTS)
    out = pl.pallas_call(
        _mlp_kernel,
        grid=(N_SUPER // BS,),
        in_specs=[
            pl.BlockSpec((BS, K, NDIM), lambda i: (i, 0, 0)),
            pl.BlockSpec((BS, NDIM), lambda i: (i, 0)),
            pl.BlockSpec((H, H), lambda i: (0, 0)),
            pl.BlockSpec((1, H), lambda i: (0, 0)),
            pl.BlockSpec((H, H), lambda i: (0, 0)),
            pl.BlockSpec((1, H), lambda i: (0, 0)),
            pl.BlockSpec((H, H), lambda i: (0, 0)),
            pl.BlockSpec((1, H), lambda i: (0, 0)),
            pl.BlockSpec((16, H), lambda i: (0, 0)),
        ],
        out_specs=pl.BlockSpec((BS, H), lambda i: (i, 0)),
        out_shape=jax.ShapeDtypeStruct((N_SUPER, H), jnp.float32),
        compiler_params=pltpu.CompilerParams(dimension_semantics=("parallel",)),
    )(neigh, q, W1, b1.reshape(1, H), W2, b2.reshape(1, H), Wp, bp.reshape(1, H), cst)

    return out[None, :, :]


# speedup vs baseline: 8.5697x; 1.3687x over previous
import numpy as np

import jax
import jax.numpy as jnp
from jax.experimental import pallas as pl
from jax.experimental.pallas import tpu as pltpu
from jax.experimental.pallas import tpu_sc as plsc

N_POINTS = 50000
N_SUPER = 1024
K = 32
H = 512
NDIM = 3
L = 128
C = 392
NP = C * L
BS = 64
PADVAL = 1.0e4
BIG = 1.0e30
BIGI = 1.0e9


def _build_consts():
    cst = np.zeros((16, H), np.float32)
    lane = np.arange(H)
    half = 64
    om = 1.0 / (10000.0 ** (np.arange(half, dtype=np.float32) / half))
    for d in range(4):
        cst[d, d * 128:(d + 1) * 128] = 1.0
    within = lane % 128
    cst[4] = om.astype(np.float32)[within % half]
    cst[5] = np.where(within < half, 0.0, np.pi / 2).astype(np.float32)
    half2 = 85
    om2 = 1.0 / (10000.0 ** (np.arange(half2, dtype=np.float32) / half2))
    for d in range(3):
        cst[6 + d, d * 170:(d + 1) * 170] = 1.0
    e = lane % 170
    valid = lane < 510
    cst[9] = np.where(valid, om2.astype(np.float32)[(e % 170) % half2], 0.0)
    cst[10] = np.where(valid & (e >= half2), np.pi / 2, 0.0).astype(np.float32)
    cst[11] = valid.astype(np.float32)
    return cst


_CONSTS = _build_consts()


def _build_sin_poly():
    xs = np.linspace(-3.36, 3.36, 6001)
    powers = np.arange(1, 15, 2)
    A = xs[:, None] ** powers[None, :]
    coef, *_ = np.linalg.lstsq(A, np.sin(xs), rcond=None)
    return coef.astype(np.float32)


_SINCOEF = _build_sin_poly()


def _psin(x):
    x2 = x * x
    p = jnp.float32(_SINCOEF[6])
    for i in range(5, -1, -1):
        p = p * x2 + jnp.float32(_SINCOEF[i])
    return p * x


def _sc_gather_rows(data, indices, window):
    num_idx = indices.shape[0]
    dim = data.shape[1]
    mesh = plsc.VectorSubcoreMesh(core_axis_name="c", subcore_axis_name="s")
    idx2 = indices.reshape(num_idx // window, window)

    @pl.kernel(out_type=jax.ShapeDtypeStruct((num_idx, dim), data.dtype),
               mesh=mesh)
    def _gather(x_hbm, i_hbm, o_hbm):
        def body(i_vmem, o_vmem):
            pltpu.sync_copy(x_hbm.at[i_vmem.at[0]], o_vmem)

        pltpu.emit_pipeline(
            body,
            grid=(num_idx // window,),
            in_specs=[pl.BlockSpec((1, window), index_map=lambda i: (i, 0))],
            out_specs=[pl.BlockSpec((window, dim), index_map=lambda i: (i, 0))],
            core_axis_name=("c", "s"),
            dimension_semantics=(pltpu.PARALLEL,),
        )(i_hbm, o_hbm)

    return _gather(data, idx2)


def _knn_kernel(q_ref, p3_ref, b_ref, idx_ref, oh_ref):
    q = q_ref[...]
    d2 = None
    for d in range(NDIM):
        t = p3_ref[d][None, :, :] - q[:, d].reshape(BS, 1, 1)
        t = t * t
        d2 = t if d2 is None else d2 + t
    m_chunk = jnp.min(d2, axis=2)
    iota_c = jax.lax.broadcasted_iota(jnp.int32, (BS, C), 1).astype(jnp.float32)
    csel = []
    for k in range(K):
        m = jnp.min(m_chunk, axis=1, keepdims=True)
        cstar = jnp.min(jnp.where(m_chunk == m, iota_c, BIGI), axis=1,
                        keepdims=True)
        oh = iota_c == cstar
        oh_ref[:, k, :] = oh.astype(jnp.float32)
        csel.append(cstar)
        m_chunk = jnp.where(oh, BIG, m_chunk)
    csel = jnp.concatenate(csel, axis=1)
    cand = jnp.dot(oh_ref[...].reshape(BS * K, C), b_ref[...],
                   preferred_element_type=jnp.float32)
    d2c = None
    for d in range(NDIM):
        cd = cand[:, d * L:(d + 1) * L].reshape(BS, K, L)
        t = cd - q[:, d].reshape(BS, 1, 1)
        t = t * t
        d2c = t if d2c is None else d2c + t
    d2c = d2c.reshape(BS, K * L)
    gidx = (csel[:, :, None] * L
            + jax.lax.broadcasted_iota(jnp.int32, (BS, K, L), 2).astype(jnp.float32)
            ).reshape(BS, K * L)
    cols = []
    for k in range(K):
        m = jnp.min(d2c, axis=1, keepdims=True)
        hit = d2c == m
        g = jnp.min(jnp.where(hit, gidx, BIGI), axis=1, keepdims=True)
        cols.append(g)
        d2c = jnp.where(hit & (gidx == g), BIG, d2c)
    idx_ref[...] = jnp.concatenate(cols, axis=1).astype(jnp.int32)


def _mlp_kernel(np_ref, q_ref, w1_ref, b1_ref, w2_ref, b2_ref, wp_ref, bp_ref,
                cst_ref, out_ref):
    cst = cst_ref[...]
    q = q_ref[...]
    neigh = np_ref[...]
    rel = neigh - q[:, None, :]
    dist = jnp.sqrt(jnp.sum(rel * rel, axis=-1, keepdims=True) + 1e-12)
    rf = jnp.concatenate([rel, dist], axis=-1).reshape(BS * K, 4)
    c_rep = None
    for d in range(4):
        t = rf[:, d:d + 1] * cst[d:d + 1, :]
        c_rep = t if c_rep is None else c_rep + t
    emb = _psin(c_rep * cst[4:5, :] + cst[5:6, :])
    x = jnp.dot(emb.astype(jnp.bfloat16), w1_ref[...],
                preferred_element_type=jnp.float32) + b1_ref[...]
    hmid = x * 0.5 * (1.0 + jax.lax.erf(x * np.float32(1.0 / np.sqrt(2.0))))
    msg = jnp.dot(hmid.astype(jnp.bfloat16), w2_ref[...],
                  preferred_element_type=jnp.float32) + b2_ref[...]
    agg = jnp.mean(msg.reshape(BS, K, H), axis=1)
    c2 = None
    for d in range(3):
        t = q[:, d:d + 1] * cst[6 + d:7 + d, :]
        c2 = t if c2 is None else c2 + t
    emb2 = _psin(c2 * cst[9:10, :] + cst[10:11, :]) * cst[11:12, :]
    out_ref[...] = (jnp.dot((agg + emb2).astype(jnp.bfloat16), wp_ref[...],
                            preferred_element_type=jnp.float32) + bp_ref[...])


def kernel(positions, W1, b1, W2, b2, Wp, bp, supernode_indices):
    pos_pad = jnp.concatenate(
        [positions, jnp.full((NP - N_POINTS, NDIM), PADVAL, jnp.float32)], axis=0)
    p3 = pos_pad.T.reshape(NDIM, C, L)
    bmat = pos_pad.reshape(C, L, NDIM).transpose(0, 2, 1).reshape(C, NDIM * L)
    pos128 = jnp.pad(positions, ((0, 0), (0, 128 - NDIM)))
    q = _sc_gather_rows(pos128, supernode_indices, 32)[:, :NDIM]

    idx = pl.pallas_call(
        _knn_kernel,
        grid=(N_SUPER // BS,),
        in_specs=[
            pl.BlockSpec((BS, NDIM), lambda i: (i, 0)),
            pl.BlockSpec((NDIM, C, L), lambda i: (0, 0, 0)),
            pl.BlockSpec((C, NDIM * L), lambda i: (0, 0)),
        ],
        out_specs=pl.BlockSpec((BS, K), lambda i: (i, 0)),
        out_shape=jax.ShapeDtypeStruct((N_SUPER, K), jnp.int32),
        scratch_shapes=[pltpu.VMEM((BS, K, C), jnp.float32)],
        compiler_params=pltpu.CompilerParams(dimension_semantics=("parallel",)),
    )(q, p3, bmat)

    neigh = _sc_gather_rows(pos128, idx.reshape(-1), 128)[:, :NDIM].reshape(
        N_SUPER, K, NDIM)

    cst = jnp.asarray(_CONSTS)
    out = pl.pallas_call(
        _mlp_kernel,
        grid=(N_SUPER // BS,),
        in_specs=[
            pl.BlockSpec((BS, K, NDIM), lambda i: (i, 0, 0)),
            pl.BlockSpec((BS, NDIM), lambda i: (i, 0)),
            pl.BlockSpec((H, H), lambda i: (0, 0)),
            pl.BlockSpec((1, H), lambda i: (0, 0)),
            pl.BlockSpec((H, H), lambda i: (0, 0)),
            pl.BlockSpec((1, H), lambda i: (0, 0)),
            pl.BlockSpec((H, H), lambda i: (0, 0)),
            pl.BlockSpec((1, H), lambda i: (0, 0)),
            pl.BlockSpec((16, H), lambda i: (0, 0)),
        ],
        out_specs=pl.BlockSpec((BS, H), lambda i: (i, 0)),
        out_shape=jax.ShapeDtypeStruct((N_SUPER, H), jnp.float32),
        compiler_params=pltpu.CompilerParams(dimension_semantics=("parallel",)),
    )(neigh, q, W1.astype(jnp.bfloat16), b1.reshape(1, H),
      W2.astype(jnp.bfloat16), b2.reshape(1, H),
      Wp.astype(jnp.bfloat16), bp.reshape(1, H), cst)

    return out[None, :, :]
